# Initial kernel scaffold; baseline (speedup 1.0000x reference)
import jax
import jax.numpy as jnp
from jax.experimental import pallas as pl


def kernel(features, edge_index, W1, b1, W2, b2, temp):
    raise NotImplementedError("write your pallas kernel here")


# speedup vs baseline: 10.9399x; 10.9399x over previous
import functools

import jax
import jax.numpy as jnp
from jax import lax
from jax.experimental import pallas as pl
from jax.experimental.pallas import tpu as pltpu
from jax.experimental.pallas import tpu_sc as plsc

N = 10000
E = 320000
F_IN = 128
CW = 48
C_REAL = 40
K_HOPS = 10

NC, NS = 2, 16
NW = NC * NS
EPC = 128
CPT = 80
EPT = CPT * EPC
E_PAD = NW * EPT
NR = 10112
RPT = NR // NS
NBUF = 4

_mesh = plsc.VectorSubcoreMesh(
    core_axis_name="c", subcore_axis_name="s", num_cores=NC, num_subcores=NS)


@functools.partial(
    pl.kernel,
    out_type=jax.ShapeDtypeStruct((NW, CPT, EPC), jnp.int32),
    mesh=_mesh,
    scratch_types=[
        pltpu.VMEM((CPT, EPC), jnp.int32),
        pltpu.VMEM((CPT, EPC), jnp.int32),
        pltpu.VMEM((CPT, EPC), jnp.int32),
    ],
)
def _sc_preproc(row_hbm, col_hbm, colp_hbm, rowv, colv, colpv):
    c = lax.axis_index("c")
    s = lax.axis_index("s")
    w = c * NS + s
    pltpu.sync_copy(row_hbm.at[w], rowv)
    pltpu.sync_copy(col_hbm.at[w], colv)

    trash = jnp.full((16,), N, jnp.int32)

    def mask_body(j, _):
        for l in range(EPC // 16):
            r = rowv[j, pl.ds(l * 16, 16)]
            cc = colv[j, pl.ds(l * 16, 16)]
            colpv[j, pl.ds(l * 16, 16)] = jnp.where(r != cc, cc, trash)
        return 0

    lax.fori_loop(0, CPT, mask_body, 0)
    pltpu.sync_copy(colpv, colp_hbm.at[w])


@functools.partial(
    pl.kernel,
    out_type=jax.ShapeDtypeStruct((NC, NR, CW), jnp.float32),
    mesh=_mesh,
    scratch_types=[
        pltpu.VMEM((CPT, EPC), jnp.int32),
        pltpu.VMEM((CPT, EPC), jnp.int32),
        pltpu.VMEM((NBUF, EPC, CW), jnp.float32),
        pltpu.VMEM_SHARED((NR, CW), jnp.float32),
        pltpu.SemaphoreType.DMA,
        pltpu.SemaphoreType.DMA,
        pltpu.SemaphoreType.DMA,
        pltpu.SemaphoreType.DMA,
        pltpu.SemaphoreType.DMA,
        pltpu.SemaphoreType.DMA,
        pltpu.SemaphoreType.DMA,
        pltpu.SemaphoreType.DMA,
    ],
    compiler_params=pltpu.CompilerParams(use_tc_tiling_on_sc=False),
)
def _sc_hop(y_hbm, row_hbm, colp_hbm, zeros_hbm,
            out_hbm, rowv, colv, bufs, acc,
            g0, g1, g2, g3, s0, s1, s2, s3):
    gsems = [g0, g1, g2, g3]
    ssems = [s0, s1, s2, s3]
    c = lax.axis_index("c")
    s = lax.axis_index("s")
    w = c * NS + s
    pltpu.sync_copy(row_hbm.at[w], rowv)
    pltpu.sync_copy(colp_hbm.at[w], colv)
    pltpu.sync_copy(zeros_hbm, acc.at[pl.ds(s * RPT, RPT)])
    plsc.subcore_barrier()

    for b in range(NBUF - 1):
        pltpu.async_copy(y_hbm.at[rowv.at[b]], bufs.at[b], gsems[b])

    def body(g, _):
        for b in range(NBUF):
            j = g * NBUF + b
            nb = (b + NBUF - 1) % NBUF

            @pl.when(j + NBUF - 1 < CPT)
            def _():
                @pl.when(j > 0)
                def _():
                    pltpu.make_async_copy(
                        bufs.at[nb], acc.at[colv.at[j]], ssems[nb]).wait()
                pltpu.async_copy(
                    y_hbm.at[rowv.at[j + NBUF - 1]], bufs.at[nb], gsems[nb])

            pltpu.make_async_copy(
                y_hbm.at[rowv.at[j]], bufs.at[b], gsems[b]).wait()
            pltpu.async_copy(
                bufs.at[b], acc.at[colv.at[j]], ssems[b], add=True)
        return 0

    lax.fori_loop(0, CPT // NBUF, body, 0)
    for b in range(NBUF):
        pltpu.make_async_copy(bufs.at[b], acc.at[colv.at[0]], ssems[b]).wait()
    plsc.subcore_barrier()
    pltpu.sync_copy(acc.at[pl.ds(s * RPT, RPT)],
                    out_hbm.at[c, pl.ds(s * RPT, RPT)])


_GRID = NS


def _mlp_body(f_ref, w1t_ref, b1_ref, w2t_ref, b2_ref, x_ref):
    h = jnp.dot(f_ref[...], w1t_ref[...], preferred_element_type=jnp.float32)
    h = jnp.maximum(h + b1_ref[...], 0.0)
    x_ref[...] = jnp.dot(h, w2t_ref[...],
                         preferred_element_type=jnp.float32) + b2_ref[...]


def _tc_mlp(f_pad, w1t, b1, w2t, b2p):
    return pl.pallas_call(
        _mlp_body,
        grid=(_GRID,),
        in_specs=[
            pl.BlockSpec((RPT, F_IN), lambda i: (i, 0)),
            pl.BlockSpec((F_IN, F_IN), lambda i: (0, 0)),
            pl.BlockSpec((1, F_IN), lambda i: (0, 0)),
            pl.BlockSpec((F_IN, CW), lambda i: (0, 0)),
            pl.BlockSpec((1, CW), lambda i: (0, 0)),
        ],
        out_specs=pl.BlockSpec((RPT, CW), lambda i: (i, 0)),
        out_shape=jax.ShapeDtypeStruct((NR, CW), jnp.float32),
    )(f_pad, w1t, b1, w2t, b2p)


def _disinit_body(p_ref, x_ref, t0_ref, dis_ref, y0_ref, h0_ref):
    deg = (p_ref[0, :, 0] + p_ref[1, :, 0] + 1.0).reshape(-1, 1)
    dis = lax.rsqrt(deg)
    x = x_ref[...]
    dis_ref[...] = dis
    y0_ref[...] = dis * x
    h0_ref[...] = t0_ref[...] * x


def _tc_disinit(partials, x, t0):
    return pl.pallas_call(
        _disinit_body,
        grid=(_GRID,),
        in_specs=[
            pl.BlockSpec((NC, RPT, CW), lambda i: (0, i, 0)),
            pl.BlockSpec((RPT, CW), lambda i: (i, 0)),
            pl.BlockSpec((1, 1), lambda i: (0, 0)),
        ],
        out_specs=[
            pl.BlockSpec((RPT, 1), lambda i: (i, 0)),
            pl.BlockSpec((RPT, CW), lambda i: (i, 0)),
            pl.BlockSpec((RPT, CW), lambda i: (i, 0)),
        ],
        out_shape=[
            jax.ShapeDtypeStruct((NR, 1), jnp.float32),
            jax.ShapeDtypeStruct((NR, CW), jnp.float32),
            jax.ShapeDtypeStruct((NR, CW), jnp.float32),
        ],
    )(partials, x, t0)


def _combine_body(p_ref, y_ref, h_ref, dis_ref, tk_ref, h1_ref, y1_ref):
    dis = dis_ref[...]
    x1 = dis * (p_ref[0] + p_ref[1] + y_ref[...])
    h1_ref[...] = h_ref[...] + tk_ref[...] * x1
    y1_ref[...] = dis * x1


def _tc_combine(partials, y, h, dis, tk):
    return pl.pallas_call(
        _combine_body,
        grid=(_GRID,),
        in_specs=[
            pl.BlockSpec((NC, RPT, CW), lambda i: (0, i, 0)),
            pl.BlockSpec((RPT, CW), lambda i: (i, 0)),
            pl.BlockSpec((RPT, CW), lambda i: (i, 0)),
            pl.BlockSpec((RPT, 1), lambda i: (i, 0)),
            pl.BlockSpec((1, 1), lambda i: (0, 0)),
        ],
        out_specs=[
            pl.BlockSpec((RPT, CW), lambda i: (i, 0)),
            pl.BlockSpec((RPT, CW), lambda i: (i, 0)),
        ],
        out_shape=[
            jax.ShapeDtypeStruct((NR, CW), jnp.float32),
            jax.ShapeDtypeStruct((NR, CW), jnp.float32),
        ],
    )(partials, y, h, dis, tk)


def _softmax_body(h_ref, ls_ref, sm_ref):
    h = h_ref[...]
    col = lax.broadcasted_iota(jnp.int32, h.shape, 1)
    hm = jnp.where(col < C_REAL, h, -1e30)
    m = jnp.max(hm, axis=1, keepdims=True)
    e = jnp.exp(hm - m)
    ssum = jnp.sum(e, axis=1, keepdims=True)
    sm_ref[...] = e / ssum
    ls_ref[...] = (hm - m) - jnp.log(ssum)


def _tc_softmax(h):
    return pl.pallas_call(
        _softmax_body,
        grid=(_GRID,),
        in_specs=[pl.BlockSpec((RPT, CW), lambda i: (i, 0))],
        out_specs=[
            pl.BlockSpec((RPT, CW), lambda i: (i, 0)),
            pl.BlockSpec((RPT, CW), lambda i: (i, 0)),
        ],
        out_shape=[
            jax.ShapeDtypeStruct((NR, CW), jnp.float32),
            jax.ShapeDtypeStruct((NR, CW), jnp.float32),
        ],
    )(h)


def kernel(features, edge_index, W1, b1, W2, b2, temp):
    f32 = jnp.float32
    row = edge_index[0]
    col = edge_index[1]
    pad = E_PAD - E
    row3 = jnp.concatenate([row, jnp.zeros((pad,), jnp.int32)]).reshape(
        NW, CPT, EPC)
    col3 = jnp.concatenate([col, jnp.zeros((pad,), jnp.int32)]).reshape(
        NW, CPT, EPC)

    zeros_cw = jnp.zeros((RPT, CW), f32)

    colp3 = _sc_preproc(row3, col3)

    f_pad = jnp.concatenate(
        [features, jnp.zeros((NR - N, F_IN), f32)], axis=0)
    w2t = jnp.concatenate(
        [W2.T, jnp.zeros((W2.shape[1], CW - C_REAL), f32)], axis=1)
    b2p = jnp.concatenate([b2, jnp.zeros((CW - C_REAL,), f32)]).reshape(1, CW)

    x = _tc_mlp(f_pad, W1.T, b1.reshape(1, F_IN), w2t, b2p)

    ones_cw = jnp.ones((NR, CW), f32)
    zeros_cw_full = jnp.zeros((NR, CW), f32)
    zeros_1 = jnp.zeros((NR, 1), f32)
    t0 = temp[0].reshape(1, 1)

    def hop_body(k, carry):
        hk, yk, disk = carry
        partials = _sc_hop(yk, row3, colp3, zeros_cw)

        def init_branch(_):
            d1, y0, h0 = _tc_disinit(partials, x, t0)
            return (h0, y0, d1)

        def step_branch(_):
            tk = lax.dynamic_slice(temp, (k,), (1,)).reshape(1, 1)
            h1, y1 = _tc_combine(partials, yk, hk, disk, tk)
            return (h1, y1, disk)

        return lax.cond(k == 0, init_branch, step_branch, 0)

    h, y, dis = lax.fori_loop(
        0, K_HOPS + 1, hop_body, (zeros_cw_full, ones_cw, zeros_1))

    ls, sm = _tc_softmax(h)
    return ls[:N, :C_REAL], sm[:N, :C_REAL]


# speedup vs baseline: 22.2766x; 2.0363x over previous
import functools

import jax
import jax.numpy as jnp
from jax import lax
from jax.experimental import pallas as pl
from jax.experimental.pallas import tpu as pltpu
from jax.experimental.pallas import tpu_sc as plsc

N = 10000
E = 320000
F_IN = 128
CW = 48
C_REAL = 40
K_HOPS = 10

NC, NS = 2, 16
NW = NC * NS
EPC = 128
CPT = 80
EPT = CPT * EPC
E_PAD = NW * EPT
NR = 10112
RPT = NR // NS
NBUF = 4

_mesh = plsc.VectorSubcoreMesh(
    core_axis_name="c", subcore_axis_name="s", num_cores=NC, num_subcores=NS)


@functools.partial(
    pl.kernel,
    out_type=jax.ShapeDtypeStruct((NW, CPT, EPC), jnp.int32),
    mesh=_mesh,
    scratch_types=[
        pltpu.VMEM((CPT, EPC), jnp.int32),
        pltpu.VMEM((CPT, EPC), jnp.int32),
        pltpu.VMEM((CPT, EPC), jnp.int32),
    ],
)
def _sc_preproc(row_hbm, col_hbm, colp_hbm, rowv, colv, colpv):
    c = lax.axis_index("c")
    s = lax.axis_index("s")
    w = c * NS + s
    pltpu.sync_copy(row_hbm.at[w], rowv)
    pltpu.sync_copy(col_hbm.at[w], colv)

    trash = jnp.full((16,), N, jnp.int32)

    def mask_body(j, _):
        for l in range(EPC // 16):
            r = rowv[j, pl.ds(l * 16, 16)]
            cc = colv[j, pl.ds(l * 16, 16)]
            colpv[j, pl.ds(l * 16, 16)] = jnp.where(r != cc, cc, trash)
        return 0

    lax.fori_loop(0, CPT, mask_body, 0)
    pltpu.sync_copy(colpv, colp_hbm.at[w])


@functools.partial(
    pl.kernel,
    out_type=jax.ShapeDtypeStruct((NC, NR, CW), jnp.float32),
    mesh=_mesh,
    scratch_types=[
        pltpu.VMEM((CPT, EPC), jnp.int32),
        pltpu.VMEM((CPT, EPC), jnp.int32),
        pltpu.VMEM((NBUF, EPC, CW), jnp.float32),
        pltpu.VMEM_SHARED((NR, CW), jnp.float32),
        pltpu.VMEM_SHARED((NR, CW), jnp.float32),
        pltpu.SemaphoreType.DMA,
        pltpu.SemaphoreType.DMA,
        pltpu.SemaphoreType.DMA,
        pltpu.SemaphoreType.DMA,
        pltpu.SemaphoreType.DMA,
        pltpu.SemaphoreType.DMA,
        pltpu.SemaphoreType.DMA,
        pltpu.SemaphoreType.DMA,
    ],
    compiler_params=pltpu.CompilerParams(use_tc_tiling_on_sc=False),
)
def _sc_hop(y_hbm, row_hbm, colp_hbm, zeros_hbm,
            out_hbm, rowv, colv, bufs, acc, yrep,
            g0, g1, g2, g3, s0, s1, s2, s3):
    gsems = [g0, g1, g2, g3]
    ssems = [s0, s1, s2, s3]
    c = lax.axis_index("c")
    s = lax.axis_index("s")
    w = c * NS + s
    pltpu.sync_copy(row_hbm.at[w], rowv)
    pltpu.sync_copy(colp_hbm.at[w], colv)
    pltpu.sync_copy(zeros_hbm, acc.at[pl.ds(s * RPT, RPT)])
    pltpu.sync_copy(y_hbm.at[pl.ds(s * RPT, RPT)], yrep.at[pl.ds(s * RPT, RPT)])
    plsc.subcore_barrier()

    for b in range(NBUF - 1):
        pltpu.async_copy(yrep.at[rowv.at[b]], bufs.at[b], gsems[b])

    def body(g, _):
        for b in range(NBUF):
            j = g * NBUF + b
            nb = (b + NBUF - 1) % NBUF

            @pl.when(j + NBUF - 1 < CPT)
            def _():
                @pl.when(j > 0)
                def _():
                    pltpu.make_async_copy(
                        bufs.at[nb], acc.at[colv.at[j]], ssems[nb]).wait()
                pltpu.async_copy(
                    yrep.at[rowv.at[j + NBUF - 1]], bufs.at[nb], gsems[nb])

            pltpu.make_async_copy(
                yrep.at[rowv.at[j]], bufs.at[b], gsems[b]).wait()
            pltpu.async_copy(
                bufs.at[b], acc.at[colv.at[j]], ssems[b], add=True)
        return 0

    lax.fori_loop(0, CPT // NBUF, body, 0)
    for b in range(NBUF):
        pltpu.make_async_copy(bufs.at[b], acc.at[colv.at[0]], ssems[b]).wait()
    plsc.subcore_barrier()
    pltpu.sync_copy(acc.at[pl.ds(s * RPT, RPT)],
                    out_hbm.at[c, pl.ds(s * RPT, RPT)])


_GRID = NS


def _mlp_body(f_ref, w1t_ref, b1_ref, w2t_ref, b2_ref, x_ref):
    h = jnp.dot(f_ref[...], w1t_ref[...], preferred_element_type=jnp.float32)
    h = jnp.maximum(h + b1_ref[...], 0.0)
    x_ref[...] = jnp.dot(h, w2t_ref[...],
                         preferred_element_type=jnp.float32) + b2_ref[...]


def _tc_mlp(f_pad, w1t, b1, w2t, b2p):
    return pl.pallas_call(
        _mlp_body,
        grid=(_GRID,),
        in_specs=[
            pl.BlockSpec((RPT, F_IN), lambda i: (i, 0)),
            pl.BlockSpec((F_IN, F_IN), lambda i: (0, 0)),
            pl.BlockSpec((1, F_IN), lambda i: (0, 0)),
            pl.BlockSpec((F_IN, CW), lambda i: (0, 0)),
            pl.BlockSpec((1, CW), lambda i: (0, 0)),
        ],
        out_specs=pl.BlockSpec((RPT, CW), lambda i: (i, 0)),
        out_shape=jax.ShapeDtypeStruct((NR, CW), jnp.float32),
    )(f_pad, w1t, b1, w2t, b2p)


def _disinit_body(p_ref, x_ref, t0_ref, dis_ref, y0_ref, h0_ref):
    deg = (p_ref[0, :, 0] + p_ref[1, :, 0] + 1.0).reshape(-1, 1)
    dis = lax.rsqrt(deg)
    x = x_ref[...]
    dis_ref[...] = dis
    y0_ref[...] = dis * x
    h0_ref[...] = t0_ref[...] * x


def _tc_disinit(partials, x, t0):
    return pl.pallas_call(
        _disinit_body,
        grid=(_GRID,),
        in_specs=[
            pl.BlockSpec((NC, RPT, CW), lambda i: (0, i, 0)),
            pl.BlockSpec((RPT, CW), lambda i: (i, 0)),
            pl.BlockSpec((1, 1), lambda i: (0, 0)),
        ],
        out_specs=[
            pl.BlockSpec((RPT, 1), lambda i: (i, 0)),
            pl.BlockSpec((RPT, CW), lambda i: (i, 0)),
            pl.BlockSpec((RPT, CW), lambda i: (i, 0)),
        ],
        out_shape=[
            jax.ShapeDtypeStruct((NR, 1), jnp.float32),
            jax.ShapeDtypeStruct((NR, CW), jnp.float32),
            jax.ShapeDtypeStruct((NR, CW), jnp.float32),
        ],
    )(partials, x, t0)


def _combine_body(p_ref, y_ref, h_ref, dis_ref, tk_ref, h1_ref, y1_ref):
    dis = dis_ref[...]
    x1 = dis * (p_ref[0] + p_ref[1] + y_ref[...])
    h1_ref[...] = h_ref[...] + tk_ref[...] * x1
    y1_ref[...] = dis * x1


def _tc_combine(partials, y, h, dis, tk):
    return pl.pallas_call(
        _combine_body,
        grid=(_GRID,),
        in_specs=[
            pl.BlockSpec((NC, RPT, CW), lambda i: (0, i, 0)),
            pl.BlockSpec((RPT, CW), lambda i: (i, 0)),
            pl.BlockSpec((RPT, CW), lambda i: (i, 0)),
            pl.BlockSpec((RPT, 1), lambda i: (i, 0)),
            pl.BlockSpec((1, 1), lambda i: (0, 0)),
        ],
        out_specs=[
            pl.BlockSpec((RPT, CW), lambda i: (i, 0)),
            pl.BlockSpec((RPT, CW), lambda i: (i, 0)),
        ],
        out_shape=[
            jax.ShapeDtypeStruct((NR, CW), jnp.float32),
            jax.ShapeDtypeStruct((NR, CW), jnp.float32),
        ],
    )(partials, y, h, dis, tk)


def _softmax_body(h_ref, ls_ref, sm_ref):
    h = h_ref[...]
    col = lax.broadcasted_iota(jnp.int32, h.shape, 1)
    hm = jnp.where(col < C_REAL, h, -1e30)
    m = jnp.max(hm, axis=1, keepdims=True)
    e = jnp.exp(hm - m)
    ssum = jnp.sum(e, axis=1, keepdims=True)
    sm_ref[...] = e / ssum
    ls_ref[...] = (hm - m) - jnp.log(ssum)


def _tc_softmax(h):
    return pl.pallas_call(
        _softmax_body,
        grid=(_GRID,),
        in_specs=[pl.BlockSpec((RPT, CW), lambda i: (i, 0))],
        out_specs=[
            pl.BlockSpec((RPT, CW), lambda i: (i, 0)),
            pl.BlockSpec((RPT, CW), lambda i: (i, 0)),
        ],
        out_shape=[
            jax.ShapeDtypeStruct((NR, CW), jnp.float32),
            jax.ShapeDtypeStruct((NR, CW), jnp.float32),
        ],
    )(h)


def kernel(features, edge_index, W1, b1, W2, b2, temp):
    f32 = jnp.float32
    row = edge_index[0]
    col = edge_index[1]
    pad = E_PAD - E
    row3 = jnp.concatenate([row, jnp.zeros((pad,), jnp.int32)]).reshape(
        NW, CPT, EPC)
    col3 = jnp.concatenate([col, jnp.zeros((pad,), jnp.int32)]).reshape(
        NW, CPT, EPC)

    zeros_cw = jnp.zeros((RPT, CW), f32)

    colp3 = _sc_preproc(row3, col3)

    f_pad = jnp.concatenate(
        [features, jnp.zeros((NR - N, F_IN), f32)], axis=0)
    w2t = jnp.concatenate(
        [W2.T, jnp.zeros((W2.shape[1], CW - C_REAL), f32)], axis=1)
    b2p = jnp.concatenate([b2, jnp.zeros((CW - C_REAL,), f32)]).reshape(1, CW)

    x = _tc_mlp(f_pad, W1.T, b1.reshape(1, F_IN), w2t, b2p)

    ones_cw = jnp.ones((NR, CW), f32)
    zeros_cw_full = jnp.zeros((NR, CW), f32)
    zeros_1 = jnp.zeros((NR, 1), f32)
    t0 = temp[0].reshape(1, 1)

    def hop_body(k, carry):
        hk, yk, disk = carry
        partials = _sc_hop(yk, row3, colp3, zeros_cw)

        def init_branch(_):
            d1, y0, h0 = _tc_disinit(partials, x, t0)
            return (h0, y0, d1)

        def step_branch(_):
            tk = lax.dynamic_slice(temp, (k,), (1,)).reshape(1, 1)
            h1, y1 = _tc_combine(partials, yk, hk, disk, tk)
            return (h1, y1, disk)

        return lax.cond(k == 0, init_branch, step_branch, 0)

    h, y, dis = lax.fori_loop(
        0, K_HOPS + 1, hop_body, (zeros_cw_full, ones_cw, zeros_1))

    ls, sm = _tc_softmax(h)
    return ls[:N, :C_REAL], sm[:N, :C_REAL]
